# Initial kernel scaffold; baseline (speedup 1.0000x reference)
import jax
import jax.numpy as jnp
from jax.experimental import pallas as pl


def kernel(x, times, pos_table, W_time, b_time):
    raise NotImplementedError("write your pallas kernel here")


# speedup vs baseline: 1.4256x; 1.4256x over previous
import jax
import jax.numpy as jnp
from jax.experimental import pallas as pl


_T_BLK = 512


def _body(times_ref, w_ref, b_ref, x_ref, pos_ref, o_ref):
    ti = pl.program_id(0)
    tt = times_ref[0, 0, pl.ds(ti * _T_BLK, _T_BLK)]
    w = w_ref[0, :]
    bb = b_ref[0, :]
    time_emb = tt[:, None] * w[None, :] + bb[None, :]
    o_ref[0] = x_ref[0] + pos_ref[...] + time_emb


def kernel(x, times, pos_table, W_time, b_time):
    B, T, D = x.shape
    n_t = T // _T_BLK
    times3 = times.reshape(B, 1, T)
    w2 = W_time.reshape(1, D)
    b2 = b_time.reshape(1, D)

    grid = (n_t, B)
    out = pl.pallas_call(
        _body,
        grid=grid,
        in_specs=[
            pl.BlockSpec((1, 1, T), lambda ti, bi: (bi, 0, 0)),
            pl.BlockSpec((1, D), lambda ti, bi: (0, 0)),
            pl.BlockSpec((1, D), lambda ti, bi: (0, 0)),
            pl.BlockSpec((1, _T_BLK, D), lambda ti, bi: (bi, ti, 0)),
            pl.BlockSpec((_T_BLK, D), lambda ti, bi: (ti, 0)),
        ],
        out_specs=pl.BlockSpec((1, _T_BLK, D), lambda ti, bi: (bi, ti, 0)),
        out_shape=jax.ShapeDtypeStruct((B, T, D), x.dtype),
    )(times3, w2, b2, x, pos_table)
    return out


# speedup vs baseline: 1.5905x; 1.1156x over previous
import jax
import jax.numpy as jnp
from jax.experimental import pallas as pl


_T_BLK = 1024


def _body(times_ref, w_ref, b_ref, x_ref, pos_ref, o_ref):
    ti = pl.program_id(0)
    tt = times_ref[0, 0, pl.ds(ti * _T_BLK, _T_BLK)]
    w = w_ref[0, :]
    bb = b_ref[0, :]
    time_emb = tt[:, None] * w[None, :] + bb[None, :]
    o_ref[0] = x_ref[0] + pos_ref[...] + time_emb


def kernel(x, times, pos_table, W_time, b_time):
    B, T, D = x.shape
    n_t = T // _T_BLK
    times3 = times.reshape(B, 1, T)
    w2 = W_time.reshape(1, D)
    b2 = b_time.reshape(1, D)

    grid = (n_t, B)
    out = pl.pallas_call(
        _body,
        grid=grid,
        in_specs=[
            pl.BlockSpec((1, 1, T), lambda ti, bi: (bi, 0, 0)),
            pl.BlockSpec((1, D), lambda ti, bi: (0, 0)),
            pl.BlockSpec((1, D), lambda ti, bi: (0, 0)),
            pl.BlockSpec((1, _T_BLK, D), lambda ti, bi: (bi, ti, 0)),
            pl.BlockSpec((_T_BLK, D), lambda ti, bi: (ti, 0)),
        ],
        out_specs=pl.BlockSpec((1, _T_BLK, D), lambda ti, bi: (bi, ti, 0)),
        out_shape=jax.ShapeDtypeStruct((B, T, D), x.dtype),
    )(times3, w2, b2, x, pos_table)
    return out


# speedup vs baseline: 1.6529x; 1.0393x over previous
import jax
import jax.numpy as jnp
from jax.experimental import pallas as pl


_T_BLK = 2048


def _body(times_ref, w_ref, b_ref, x_ref, pos_ref, o_ref):
    ti = pl.program_id(0)
    tt = times_ref[0, 0, pl.ds(ti * _T_BLK, _T_BLK)]
    w = w_ref[0, :]
    bb = b_ref[0, :]
    time_emb = tt[:, None] * w[None, :] + bb[None, :]
    o_ref[0] = x_ref[0] + pos_ref[...] + time_emb


def kernel(x, times, pos_table, W_time, b_time):
    B, T, D = x.shape
    n_t = T // _T_BLK
    times3 = times.reshape(B, 1, T)
    w2 = W_time.reshape(1, D)
    b2 = b_time.reshape(1, D)

    grid = (n_t, B)
    out = pl.pallas_call(
        _body,
        grid=grid,
        in_specs=[
            pl.BlockSpec((1, 1, T), lambda ti, bi: (bi, 0, 0)),
            pl.BlockSpec((1, D), lambda ti, bi: (0, 0)),
            pl.BlockSpec((1, D), lambda ti, bi: (0, 0)),
            pl.BlockSpec((1, _T_BLK, D), lambda ti, bi: (bi, ti, 0)),
            pl.BlockSpec((_T_BLK, D), lambda ti, bi: (ti, 0)),
        ],
        out_specs=pl.BlockSpec((1, _T_BLK, D), lambda ti, bi: (bi, ti, 0)),
        out_shape=jax.ShapeDtypeStruct((B, T, D), x.dtype),
    )(times3, w2, b2, x, pos_table)
    return out
